# baseline (speedup 1.0000x reference)
import jax
import jax.numpy as jnp
from jax import lax
from jax.experimental import pallas as pl
from jax.experimental.pallas import tpu as pltpu
from jax.experimental.pallas import tpu_sc as plsc

N = 10000
E = 320000
H = 128
G = 64
EPS = 1e-5

NC = 2
NS = 16
NW = NC * NS
CHUNK = 64
NCHUNK = 160
EPW_PAD = NCHUNK * CHUNK
ACC_ROWS = 10240
TRASH = N + 16
ROWS_PER_SUB = ACC_ROWS // NS
BLK = 16
NBLK = NCHUNK // BLK

def _wid():
    return lax.axis_index("s") * NC + lax.axis_index("c")


import functools


@functools.lru_cache(maxsize=None)
def _sc_calls():
    mesh = plsc.VectorSubcoreMesh(
        core_axis_name="c", subcore_axis_name="s",
        num_cores=NC, num_subcores=NS)
    spmm_call = pl.kernel(
        _spmm_body,
        out_type=jax.ShapeDtypeStruct((NC, ACC_ROWS, H), jnp.float32),
        mesh=mesh,
        scratch_types=[
            pltpu.VMEM((BLK, CHUNK), jnp.int32),
            pltpu.VMEM((BLK, CHUNK), jnp.int32),
            pltpu.VMEM((BLK, CHUNK), jnp.int32),
            pltpu.VMEM((BLK, CHUNK), jnp.int32),
            pltpu.VMEM((CHUNK, H), jnp.float32),
            pltpu.VMEM((CHUNK, H), jnp.float32),
            pltpu.VMEM((CHUNK, H), jnp.float32),
            pltpu.VMEM((CHUNK, H), jnp.float32),
            pltpu.VMEM_SHARED((ACC_ROWS, H), jnp.float32),
            pltpu.SemaphoreType.DMA,
            pltpu.SemaphoreType.DMA,
            pltpu.SemaphoreType.DMA,
            pltpu.SemaphoreType.DMA,
            pltpu.SemaphoreType.DMA,
            pltpu.SemaphoreType.DMA,
            pltpu.SemaphoreType.DMA,
            pltpu.SemaphoreType.DMA,
            pltpu.SemaphoreType.DMA,
        ],
    )
    return spmm_call


def _spmm_body(hs_hbm, srcs_hbm, dsts_hbm, out_hbm,
               s0v, s1v, d0v, d1v, r0, r1, r2, r3, acc,
               g0, g1, g2, g3, ss0, ss1, ss2, ss3, isem):
    cid = lax.axis_index("c")
    sid = lax.axis_index("s")
    w = _wid()
    sbufs = (s0v, s1v)
    dbufs = (d0v, d1v)
    rows = (r0, r1, r2, r3)
    gsem = (g0, g1, g2, g3)
    ssem = (ss0, ss1, ss2, ss3)

    def zrow(r, _):
        for c in range(H // 16):
            r0[r, pl.ds(c * 16, 16)] = jnp.zeros((16,), jnp.float32)
        return 0
    lax.fori_loop(0, CHUNK, zrow, 0)
    base = sid * ROWS_PER_SUB
    for k in range(ROWS_PER_SUB // CHUNK):
        pltpu.sync_copy(r0, acc.at[pl.ds(base + k * CHUNK, CHUNK)])
    plsc.subcore_barrier()

    def start_blk(b, i):
        d1 = pltpu.async_copy(srcs_hbm.at[w, pl.ds(b * BLK, BLK)],
                              sbufs[i], isem)
        d2 = pltpu.async_copy(dsts_hbm.at[w, pl.ds(b * BLK, BLK)],
                              dbufs[i], isem)
        return d1, d2

    def drain_scatters(dv):
        for t in range(4):
            pltpu.make_async_copy(rows[t], acc.at[dv.at[t]], ssem[t]).wait()

    pend = start_blk(0, 0)
    for b in range(NBLK):
        i = b % 2
        for dsc in pend:
            dsc.wait()
        sv, dv = sbufs[i], dbufs[i]
        if b > 0:
            drain_scatters(dv)
        if b + 1 < NBLK:
            pend = start_blk(b + 1, 1 - i)

        def quad(qq, _):
            jb = qq * 4
            descs = []
            for t in range(4):
                @pl.when(qq > 0)
                def _drain(t=t):
                    pltpu.make_async_copy(
                        rows[t], acc.at[dv.at[jb + t]], ssem[t]).wait()
                descs.append(
                    pltpu.async_copy(hs_hbm.at[sv.at[jb + t]], rows[t],
                                     gsem[t]))
            for t in range(4):
                descs[t].wait()
                pltpu.async_copy(rows[t], acc.at[dv.at[jb + t]], ssem[t],
                                 add=True)
            return 0
        lax.fori_loop(0, BLK // 4, quad, 0)
    drain_scatters(dbufs[(NBLK - 1) % 2])
    plsc.subcore_barrier()

    pltpu.sync_copy(acc.at[pl.ds(base, ROWS_PER_SUB)],
                    out_hbm.at[cid, pl.ds(base, ROWS_PER_SUB)])


def _lrelu(x):
    return jnp.where(x > 0, x, 0.1 * x)


def _bn_cols(x, g, be):
    m = jnp.mean(x, axis=0, keepdims=True)
    c = x - m
    v = jnp.mean(c * c, axis=0, keepdims=True)
    return c * lax.rsqrt(v + EPS) * g + be


def _prep_body(x_ref, w_ref, d0_ref, d1_ref, hs_ref, dis_ref):
    deg = 1.0 + d0_ref[...][:N, 0:1] + d1_ref[...][:N, 0:1]
    dis = lax.rsqrt(deg)
    dis_ref[...] = dis
    hs_ref[...] = jnp.dot(x_ref[...], w_ref[...],
                          preferred_element_type=jnp.float32) * dis


_prep_call = pl.pallas_call(
    _prep_body,
    out_shape=(jax.ShapeDtypeStruct((N, H), jnp.float32),
               jax.ShapeDtypeStruct((N, 1), jnp.float32)),
)


def _mid_body(a0, a1, hsp, dis, b, g, be, w, out):
    dis_v = dis[...]
    agg = (a0[...][:N] + a1[...][:N] + hsp[...]) * dis_v + b[...]
    h = _lrelu(_bn_cols(agg, g[...], be[...]))
    out[...] = jnp.dot(h, w[...], preferred_element_type=jnp.float32) * dis_v


_mid_call = pl.pallas_call(
    _mid_body,
    out_shape=jax.ShapeDtypeStruct((N, H), jnp.float32),
)


def _final_body(a0, a1, hsp, dis, b, g, be, bt_ref, fw1, fb1, g4, be4,
                fw2, fb2, out):
    agg = (a0[...][:N] + a1[...][:N] + hsp[...]) * dis[...] + b[...]
    h = _lrelu(_bn_cols(agg, g[...], be[...]))
    gi = lax.broadcasted_iota(jnp.int32, (G, N), 0)
    msk = (bt_ref[...] == gi).astype(jnp.float32)
    s = jnp.dot(msk, h, preferred_element_type=jnp.float32, precision=lax.Precision.HIGHEST)
    cnt = jnp.sum(msk, axis=1, keepdims=True)
    pooled = s / jnp.maximum(cnt, 1.0)
    z = jnp.dot(pooled, fw1[...], preferred_element_type=jnp.float32) + fb1[...]
    zl = _lrelu(_bn_cols(z, g4[...], be4[...]))
    out[...] = jnp.dot(zl, fw2[...], preferred_element_type=jnp.float32) + fb2[...]


_final_call = pl.pallas_call(
    _final_body,
    out_shape=jax.ShapeDtypeStruct((G, H), jnp.float32),
)


def kernel(x, W1, b1, g1, be1, W2, b2, g2, be2, W3, b3, g3, be3,
           fcW1, fcb1, g4, be4, fcW2, fcb2, edge_index, batch):
    src = edge_index[0]
    dst = edge_index[1]
    pad = EPW_PAD * NW - E
    srcs = jnp.concatenate(
        [src, jnp.zeros((pad,), jnp.int32)]).reshape(NW, NCHUNK, CHUNK)
    dsts = jnp.concatenate(
        [dst, jnp.full((pad,), TRASH, jnp.int32)]).reshape(NW, NCHUNK, CHUNK)

    _spmm_call = _sc_calls()
    deg = _spmm_call(jnp.ones((N, H), jnp.float32),
                     jnp.zeros_like(srcs), dsts)
    hs1, dis = _prep_call(x, W1, deg[0], deg[1])

    r2 = lambda v: v.reshape(1, -1)
    p1 = _spmm_call(hs1, srcs, dsts)
    hs2 = _mid_call(p1[0], p1[1], hs1, dis, r2(b1), r2(g1), r2(be1), W2)
    p2 = _spmm_call(hs2, srcs, dsts)
    hs3 = _mid_call(p2[0], p2[1], hs2, dis, r2(b2), r2(g2), r2(be2), W3)
    p3 = _spmm_call(hs3, srcs, dsts)
    out = _final_call(p3[0], p3[1], hs3, dis, r2(b3), r2(g3), r2(be3),
                      batch.reshape(1, N), fcW1, r2(fcb1), r2(g4), r2(be4),
                      fcW2, r2(fcb2))
    return out


# speedup vs baseline: 1.0023x; 1.0023x over previous
import jax
import jax.numpy as jnp
from jax import lax
from jax.experimental import pallas as pl
from jax.experimental.pallas import tpu as pltpu
from jax.experimental.pallas import tpu_sc as plsc

N = 10000
E = 320000
H = 128
G = 64
EPS = 1e-5

NC = 2
NS = 16
NW = NC * NS
CHUNK = 128
NCHUNK = 80
EPW_PAD = NCHUNK * CHUNK
ACC_ROWS = 10240
TRASH = N + 16
ROWS_PER_SUB = ACC_ROWS // NS
BLK = 8
NBLK = NCHUNK // BLK

def _wid():
    return lax.axis_index("s") * NC + lax.axis_index("c")


import functools


@functools.lru_cache(maxsize=None)
def _sc_calls():
    mesh = plsc.VectorSubcoreMesh(
        core_axis_name="c", subcore_axis_name="s",
        num_cores=NC, num_subcores=NS)
    spmm_call = pl.kernel(
        _spmm_body,
        out_type=jax.ShapeDtypeStruct((NC, ACC_ROWS, H), jnp.float32),
        mesh=mesh,
        scratch_types=[
            pltpu.VMEM((BLK, CHUNK), jnp.int32),
            pltpu.VMEM((BLK, CHUNK), jnp.int32),
            pltpu.VMEM((BLK, CHUNK), jnp.int32),
            pltpu.VMEM((BLK, CHUNK), jnp.int32),
            pltpu.VMEM((CHUNK, H), jnp.float32),
            pltpu.VMEM((CHUNK, H), jnp.float32),
            pltpu.VMEM_SHARED((ACC_ROWS, H), jnp.float32),
            pltpu.SemaphoreType.DMA,
            pltpu.SemaphoreType.DMA,
            pltpu.SemaphoreType.DMA,
        ],
    )
    return spmm_call


def _spmm_body(hs_hbm, srcs_hbm, dsts_hbm, out_hbm,
               s0v, s1v, d0v, d1v, r0, r1, acc, g0, g1, isem):
    cid = lax.axis_index("c")
    sid = lax.axis_index("s")
    w = _wid()
    sbufs = (s0v, s1v)
    dbufs = (d0v, d1v)
    rows = (r0, r1)
    gsem = (g0, g1)

    def zrow(r, _):
        for c in range(H // 16):
            r0[r, pl.ds(c * 16, 16)] = jnp.zeros((16,), jnp.float32)
        return 0
    lax.fori_loop(0, CHUNK, zrow, 0)
    base = sid * ROWS_PER_SUB
    for k in range(ROWS_PER_SUB // CHUNK):
        pltpu.sync_copy(r0, acc.at[pl.ds(base + k * CHUNK, CHUNK)])
    plsc.subcore_barrier()

    def start_blk(b, i):
        d1 = pltpu.async_copy(srcs_hbm.at[w, pl.ds(b * BLK, BLK)],
                              sbufs[i], isem)
        d2 = pltpu.async_copy(dsts_hbm.at[w, pl.ds(b * BLK, BLK)],
                              dbufs[i], isem)
        return d1, d2

    pend = start_blk(0, 0)
    for b in range(NBLK):
        i = b % 2
        for dsc in pend:
            dsc.wait()
        if b + 1 < NBLK:
            pend = start_blk(b + 1, 1 - i)
        sv, dv = sbufs[i], dbufs[i]

        def pair(qq, _):
            jb = qq * 2
            descs = [
                pltpu.async_copy(hs_hbm.at[sv.at[jb + t]], rows[t], gsem[t])
                for t in range(2)
            ]
            for t in range(2):
                descs[t].wait()
                pltpu.sync_copy(rows[t], acc.at[dv.at[jb + t]], add=True)
            return 0
        lax.fori_loop(0, BLK // 2, pair, 0)
    plsc.subcore_barrier()

    pltpu.sync_copy(acc.at[pl.ds(base, ROWS_PER_SUB)],
                    out_hbm.at[cid, pl.ds(base, ROWS_PER_SUB)])


def _lrelu(x):
    return jnp.where(x > 0, x, 0.1 * x)


def _bn_cols(x, g, be):
    m = jnp.mean(x, axis=0, keepdims=True)
    c = x - m
    v = jnp.mean(c * c, axis=0, keepdims=True)
    return c * lax.rsqrt(v + EPS) * g + be


def _prep_body(x_ref, w_ref, d0_ref, d1_ref, hs_ref, dis_ref):
    deg = 1.0 + d0_ref[...][:N, 0:1] + d1_ref[...][:N, 0:1]
    dis = lax.rsqrt(deg)
    dis_ref[...] = dis
    hs_ref[...] = jnp.dot(x_ref[...], w_ref[...],
                          preferred_element_type=jnp.float32) * dis


_prep_call = pl.pallas_call(
    _prep_body,
    out_shape=(jax.ShapeDtypeStruct((N, H), jnp.float32),
               jax.ShapeDtypeStruct((N, 1), jnp.float32)),
)


def _mid_body(a0, a1, hsp, dis, b, g, be, w, out):
    dis_v = dis[...]
    agg = (a0[...][:N] + a1[...][:N] + hsp[...]) * dis_v + b[...]
    h = _lrelu(_bn_cols(agg, g[...], be[...]))
    out[...] = jnp.dot(h, w[...], preferred_element_type=jnp.float32) * dis_v


_mid_call = pl.pallas_call(
    _mid_body,
    out_shape=jax.ShapeDtypeStruct((N, H), jnp.float32),
)


def _final_body(a0, a1, hsp, dis, b, g, be, bt_ref, fw1, fb1, g4, be4,
                fw2, fb2, out):
    agg = (a0[...][:N] + a1[...][:N] + hsp[...]) * dis[...] + b[...]
    h = _lrelu(_bn_cols(agg, g[...], be[...]))
    gi = lax.broadcasted_iota(jnp.int32, (G, N), 0)
    msk = (bt_ref[...] == gi).astype(jnp.float32)
    s = jnp.dot(msk, h, preferred_element_type=jnp.float32, precision=lax.Precision.HIGHEST)
    cnt = jnp.sum(msk, axis=1, keepdims=True)
    pooled = s / jnp.maximum(cnt, 1.0)
    z = jnp.dot(pooled, fw1[...], preferred_element_type=jnp.float32) + fb1[...]
    zl = _lrelu(_bn_cols(z, g4[...], be4[...]))
    out[...] = jnp.dot(zl, fw2[...], preferred_element_type=jnp.float32) + fb2[...]


_final_call = pl.pallas_call(
    _final_body,
    out_shape=jax.ShapeDtypeStruct((G, H), jnp.float32),
)


def kernel(x, W1, b1, g1, be1, W2, b2, g2, be2, W3, b3, g3, be3,
           fcW1, fcb1, g4, be4, fcW2, fcb2, edge_index, batch):
    src = edge_index[0]
    dst = edge_index[1]
    pad = EPW_PAD * NW - E
    srcs = jnp.concatenate(
        [src, jnp.zeros((pad,), jnp.int32)]).reshape(NW, NCHUNK, CHUNK)
    dsts = jnp.concatenate(
        [dst, jnp.full((pad,), TRASH, jnp.int32)]).reshape(NW, NCHUNK, CHUNK)

    _spmm_call = _sc_calls()
    deg = _spmm_call(jnp.ones((N, H), jnp.float32),
                     jnp.zeros_like(srcs), dsts)
    hs1, dis = _prep_call(x, W1, deg[0], deg[1])

    r2 = lambda v: v.reshape(1, -1)
    p1 = _spmm_call(hs1, srcs, dsts)
    hs2 = _mid_call(p1[0], p1[1], hs1, dis, r2(b1), r2(g1), r2(be1), W2)
    p2 = _spmm_call(hs2, srcs, dsts)
    hs3 = _mid_call(p2[0], p2[1], hs2, dis, r2(b2), r2(g2), r2(be2), W3)
    p3 = _spmm_call(hs3, srcs, dsts)
    out = _final_call(p3[0], p3[1], hs3, dis, r2(b3), r2(g3), r2(be3),
                      batch.reshape(1, N), fcW1, r2(fcb1), r2(g4), r2(be4),
                      fcW2, r2(fcb2))
    return out


# speedup vs baseline: 5.7499x; 5.7369x over previous
import jax
import jax.numpy as jnp
from jax import lax
from jax.experimental import pallas as pl
from jax.experimental.pallas import tpu as pltpu
from jax.experimental.pallas import tpu_sc as plsc

N = 10000
E = 320000
H = 128
G = 64
EPS = 1e-5

NC = 2
NS = 16
NW = NC * NS
CHUNK = 128
NCHUNK = 80
EPW_PAD = NCHUNK * CHUNK
ACC_ROWS = 10240
TRASH = N + 16
ROWS_PER_SUB = ACC_ROWS // NS
BLK = 8
NBLK = NCHUNK // BLK

def _wid():
    return lax.axis_index("s") * NC + lax.axis_index("c")


import functools


@functools.lru_cache(maxsize=None)
def _sc_calls():
    mesh = plsc.VectorSubcoreMesh(
        core_axis_name="c", subcore_axis_name="s",
        num_cores=NC, num_subcores=NS)
    spmm_call = pl.kernel(
        _spmm_body,
        out_type=jax.ShapeDtypeStruct((NC, ACC_ROWS, H), jnp.float32),
        mesh=mesh,
        scratch_types=[
            pltpu.VMEM((BLK, CHUNK), jnp.int32),
            pltpu.VMEM((BLK, CHUNK), jnp.int32),
            pltpu.VMEM((BLK, CHUNK), jnp.int32),
            pltpu.VMEM((BLK, CHUNK), jnp.int32),
            pltpu.VMEM((CHUNK, H), jnp.float32),
            pltpu.VMEM((CHUNK, H), jnp.float32),
            pltpu.VMEM_SHARED((ACC_ROWS, H), jnp.float32),
            pltpu.SemaphoreType.DMA,
            pltpu.SemaphoreType.DMA,
            pltpu.SemaphoreType.DMA,
        ],
    )
    return spmm_call


def _spmm_body(hs_hbm, srcs_hbm, dsts_hbm, out_hbm,
               s0v, s1v, d0v, d1v, r0, r1, acc, g0, g1, isem):
    cid = lax.axis_index("c")
    sid = lax.axis_index("s")
    w = _wid()
    sbufs = (s0v, s1v)
    dbufs = (d0v, d1v)
    rows = (r0, r1)
    gsem = (g0, g1)

    def zrow(r, _):
        for c in range(H // 16):
            r0[r, pl.ds(c * 16, 16)] = jnp.zeros((16,), jnp.float32)
        return 0
    lax.fori_loop(0, CHUNK, zrow, 0)
    base = sid * ROWS_PER_SUB
    for k in range(ROWS_PER_SUB // CHUNK):
        pltpu.sync_copy(r0, acc.at[pl.ds(base + k * CHUNK, CHUNK)])
    plsc.subcore_barrier()

    def start_blk(b, i):
        d1 = pltpu.async_copy(srcs_hbm.at[w, pl.ds(b * BLK, BLK)],
                              sbufs[i], isem)
        d2 = pltpu.async_copy(dsts_hbm.at[w, pl.ds(b * BLK, BLK)],
                              dbufs[i], isem)
        return d1, d2

    pend = start_blk(0, 0)
    for b in range(NBLK):
        i = b % 2
        for dsc in pend:
            dsc.wait()
        if b + 1 < NBLK:
            pend = start_blk(b + 1, 1 - i)
        sv, dv = sbufs[i], dbufs[i]

        def pair(qq, _):
            jb = qq * 2
            descs = [
                pltpu.async_copy(hs_hbm.at[sv.at[jb + t]], rows[t], gsem[t])
                for t in range(2)
            ]
            for t in range(2):
                descs[t].wait()
                pltpu.sync_copy(rows[t], acc.at[dv.at[jb + t]], add=True)
            return 0
        lax.fori_loop(0, BLK // 2, pair, 0)
    plsc.subcore_barrier()

    pltpu.sync_copy(acc.at[pl.ds(base, ROWS_PER_SUB)],
                    out_hbm.at[cid, pl.ds(base, ROWS_PER_SUB)])


def _lrelu(x):
    return jnp.where(x > 0, x, 0.1 * x)


def _bn_cols(x, g, be):
    m = jnp.mean(x, axis=0, keepdims=True)
    c = x - m
    v = jnp.mean(c * c, axis=0, keepdims=True)
    return c * lax.rsqrt(v + EPS) * g + be


def _prep_body(x_ref, w_ref, d0_ref, d1_ref, hs_ref, dis_ref):
    deg = 1.0 + d0_ref[...][:N, 0:1] + d1_ref[...][:N, 0:1]
    dis = lax.rsqrt(deg)
    dis_ref[...] = dis
    hs_ref[...] = jnp.dot(x_ref[...], w_ref[...],
                          preferred_element_type=jnp.float32) * dis


_prep_call = pl.pallas_call(
    _prep_body,
    out_shape=(jax.ShapeDtypeStruct((N, H), jnp.float32),
               jax.ShapeDtypeStruct((N, 1), jnp.float32)),
)


def _mid_body(a0, a1, hsp, dis, b, g, be, w, out):
    dis_v = dis[...]
    agg = (a0[...][:N] + a1[...][:N] + hsp[...]) * dis_v + b[...]
    h = _lrelu(_bn_cols(agg, g[...], be[...]))
    out[...] = jnp.dot(h, w[...], preferred_element_type=jnp.float32) * dis_v


_mid_call = pl.pallas_call(
    _mid_body,
    out_shape=jax.ShapeDtypeStruct((N, H), jnp.float32),
)


def _final_body(a0, a1, hsp, dis, b, g, be, bt_ref, fw1, fb1, g4, be4,
                fw2, fb2, out):
    agg = (a0[...][:N] + a1[...][:N] + hsp[...]) * dis[...] + b[...]
    h = _lrelu(_bn_cols(agg, g[...], be[...]))
    gi = lax.broadcasted_iota(jnp.int32, (G, N), 0)
    msk = (bt_ref[...] == gi).astype(jnp.float32)
    s = jnp.dot(msk, h, preferred_element_type=jnp.float32, precision=lax.Precision.HIGHEST)
    cnt = jnp.sum(msk, axis=1, keepdims=True)
    pooled = s / jnp.maximum(cnt, 1.0)
    z = jnp.dot(pooled, fw1[...], preferred_element_type=jnp.float32) + fb1[...]
    zl = _lrelu(_bn_cols(z, g4[...], be4[...]))
    out[...] = jnp.dot(zl, fw2[...], preferred_element_type=jnp.float32) + fb2[...]


_final_call = pl.pallas_call(
    _final_body,
    out_shape=jax.ShapeDtypeStruct((G, H), jnp.float32),
)


def kernel(x, W1, b1, g1, be1, W2, b2, g2, be2, W3, b3, g3, be3,
           fcW1, fcb1, g4, be4, fcW2, fcb2, edge_index, batch):
    src = edge_index[0]
    dst = edge_index[1]
    pad = EPW_PAD * NW - E
    srcs = jnp.concatenate(
        [src, jnp.zeros((pad,), jnp.int32)]).reshape(NW, NCHUNK, CHUNK)
    dsts = jnp.concatenate(
        [dst, jnp.full((pad,), TRASH, jnp.int32)]).reshape(NW, NCHUNK, CHUNK)

    _spmm_call = _sc_calls()
    deg = _spmm_call(jnp.ones((N, H), jnp.float32), srcs, dsts)
    hs1, dis = _prep_call(x, W1, deg[0], deg[1])

    r2 = lambda v: v.reshape(1, -1)
    p1 = _spmm_call(hs1, srcs, dsts)
    hs2 = _mid_call(p1[0], p1[1], hs1, dis, r2(b1), r2(g1), r2(be1), W2)
    p2 = _spmm_call(hs2, srcs, dsts)
    hs3 = _mid_call(p2[0], p2[1], hs2, dis, r2(b2), r2(g2), r2(be2), W3)
    p3 = _spmm_call(hs3, srcs, dsts)
    out = _final_call(p3[0], p3[1], hs3, dis, r2(b3), r2(g3), r2(be3),
                      batch.reshape(1, N), fcW1, r2(fcb1), r2(g4), r2(be4),
                      fcW2, r2(fcb2))
    return out


# speedup vs baseline: 6.3634x; 1.1067x over previous
import jax
import jax.numpy as jnp
from jax import lax
from jax.experimental import pallas as pl
from jax.experimental.pallas import tpu as pltpu
from jax.experimental.pallas import tpu_sc as plsc

N = 10000
E = 320000
H = 128
G = 64
EPS = 1e-5

NC = 2
NS = 16
NW = NC * NS
CHUNK = 64
BLK = 16
NBLK0 = 6
NBLK1 = 14
C0 = NBLK0 * BLK
C1 = NBLK1 * BLK
C0TOT = NS * C0
TOT_CHUNKS = NS * (C0 + C1)
EPAD = TOT_CHUNKS * CHUNK
ACC_ROWS = 10240
TRASH = N + 16
ROWS_PER_SUB = ACC_ROWS // NS

import functools


@functools.lru_cache(maxsize=None)
def _sc_calls():
    mesh = plsc.VectorSubcoreMesh(
        core_axis_name="c", subcore_axis_name="s",
        num_cores=NC, num_subcores=NS)
    spmm_call = pl.kernel(
        _spmm_body,
        out_type=jax.ShapeDtypeStruct((NC, ACC_ROWS, H), jnp.float32),
        mesh=mesh,
        scratch_types=[
            pltpu.VMEM((BLK, CHUNK), jnp.int32),
            pltpu.VMEM((BLK, CHUNK), jnp.int32),
            pltpu.VMEM((CHUNK, H), jnp.float32),
            pltpu.VMEM((CHUNK, H), jnp.float32),
            pltpu.VMEM_SHARED((ACC_ROWS, H), jnp.float32),
            pltpu.SemaphoreType.DMA,
            pltpu.SemaphoreType.DMA,
        ],
    )
    return spmm_call


def _spmm_body(hs_hbm, srcs_hbm, dsts_hbm, out_hbm,
               sv, dv, r0, r1, acc, g0, g1):
    cid = lax.axis_index("c")
    sid = lax.axis_index("s")
    rows = (r0, r1)
    gsem = (g0, g1)

    def zrow(r, _):
        for c in range(H // 16):
            r0[r, pl.ds(c * 16, 16)] = jnp.zeros((16,), jnp.float32)
        return 0
    lax.fori_loop(0, CHUNK, zrow, 0)
    base = sid * ROWS_PER_SUB
    for k in range(ROWS_PER_SUB // CHUNK):
        pltpu.sync_copy(r0, acc.at[pl.ds(base + k * CHUNK, CHUNK)])
    plsc.subcore_barrier()

    nblk = jnp.where(cid == 0, NBLK0, NBLK1)
    start = jnp.where(cid == 0, sid * C0, C0TOT + sid * C1)

    def blk(b, _):
        cbase = pl.multiple_of(start + b * BLK, 8)
        pltpu.sync_copy(srcs_hbm.at[pl.ds(cbase, BLK)], sv)
        pltpu.sync_copy(dsts_hbm.at[pl.ds(cbase, BLK)], dv)

        def pair(qq, _):
            jb = qq * 2
            descs = [
                pltpu.async_copy(hs_hbm.at[sv.at[jb + t]], rows[t], gsem[t])
                for t in range(2)
            ]
            for t in range(2):
                descs[t].wait()
                pltpu.sync_copy(rows[t], acc.at[dv.at[jb + t]], add=True)
            return 0
        lax.fori_loop(0, BLK // 2, pair, 0)
        return 0
    lax.fori_loop(0, nblk, blk, 0)
    plsc.subcore_barrier()

    pltpu.sync_copy(acc.at[pl.ds(base, ROWS_PER_SUB)],
                    out_hbm.at[cid, pl.ds(base, ROWS_PER_SUB)])


def _lrelu(x):
    return jnp.where(x > 0, x, 0.1 * x)


def _bn_cols(x, g, be):
    m = jnp.mean(x, axis=0, keepdims=True)
    c = x - m
    v = jnp.mean(c * c, axis=0, keepdims=True)
    return c * lax.rsqrt(v + EPS) * g + be


def _prep_body(x_ref, w_ref, d0_ref, d1_ref, hs_ref, dis_ref):
    deg = 1.0 + d0_ref[...][:N, 0:1] + d1_ref[...][:N, 0:1]
    dis = lax.rsqrt(deg)
    dis_ref[...] = dis
    hs_ref[...] = jnp.dot(x_ref[...], w_ref[...],
                          preferred_element_type=jnp.float32) * dis


_prep_call = pl.pallas_call(
    _prep_body,
    out_shape=(jax.ShapeDtypeStruct((N, H), jnp.float32),
               jax.ShapeDtypeStruct((N, 1), jnp.float32)),
)


def _mid_body(a0, a1, hsp, dis, b, g, be, w, out):
    dis_v = dis[...]
    agg = (a0[...][:N] + a1[...][:N] + hsp[...]) * dis_v + b[...]
    h = _lrelu(_bn_cols(agg, g[...], be[...]))
    out[...] = jnp.dot(h, w[...], preferred_element_type=jnp.float32) * dis_v


_mid_call = pl.pallas_call(
    _mid_body,
    out_shape=jax.ShapeDtypeStruct((N, H), jnp.float32),
)


def _final_body(a0, a1, hsp, dis, b, g, be, bt_ref, fw1, fb1, g4, be4,
                fw2, fb2, out):
    agg = (a0[...][:N] + a1[...][:N] + hsp[...]) * dis[...] + b[...]
    h = _lrelu(_bn_cols(agg, g[...], be[...]))
    gi = lax.broadcasted_iota(jnp.int32, (G, N), 0)
    msk = (bt_ref[...] == gi).astype(jnp.float32)
    s = jnp.dot(msk, h, preferred_element_type=jnp.float32, precision=lax.Precision.HIGHEST)
    cnt = jnp.sum(msk, axis=1, keepdims=True)
    pooled = s / jnp.maximum(cnt, 1.0)
    z = jnp.dot(pooled, fw1[...], preferred_element_type=jnp.float32) + fb1[...]
    zl = _lrelu(_bn_cols(z, g4[...], be4[...]))
    out[...] = jnp.dot(zl, fw2[...], preferred_element_type=jnp.float32) + fb2[...]


_final_call = pl.pallas_call(
    _final_body,
    out_shape=jax.ShapeDtypeStruct((G, H), jnp.float32),
)


def kernel(x, W1, b1, g1, be1, W2, b2, g2, be2, W3, b3, g3, be3,
           fcW1, fcb1, g4, be4, fcW2, fcb2, edge_index, batch):
    src = edge_index[0]
    dst = edge_index[1]
    pad = EPAD - E
    srcs = jnp.concatenate(
        [src, jnp.zeros((pad,), jnp.int32)]).reshape(TOT_CHUNKS, CHUNK)
    dsts = jnp.concatenate(
        [dst, jnp.full((pad,), TRASH, jnp.int32)]).reshape(TOT_CHUNKS, CHUNK)

    _spmm_call = _sc_calls()
    deg = _spmm_call(jnp.ones((N, H), jnp.float32), srcs, dsts)
    hs1, dis = _prep_call(x, W1, deg[0], deg[1])

    r2 = lambda v: v.reshape(1, -1)
    p1 = _spmm_call(hs1, srcs, dsts)
    hs2 = _mid_call(p1[0], p1[1], hs1, dis, r2(b1), r2(g1), r2(be1), W2)
    p2 = _spmm_call(hs2, srcs, dsts)
    hs3 = _mid_call(p2[0], p2[1], hs2, dis, r2(b2), r2(g2), r2(be2), W3)
    p3 = _spmm_call(hs3, srcs, dsts)
    out = _final_call(p3[0], p3[1], hs3, dis, r2(b3), r2(g3), r2(be3),
                      batch.reshape(1, N), fcW1, r2(fcb1), r2(g4), r2(be4),
                      fcW2, r2(fcb2))
    return out


# speedup vs baseline: 7.7623x; 1.2198x over previous
import jax
import jax.numpy as jnp
from jax import lax
from jax.experimental import pallas as pl
from jax.experimental.pallas import tpu as pltpu
from jax.experimental.pallas import tpu_sc as plsc

N = 10000
E = 320000
H = 128
G = 64
EPS = 1e-5

NC = 2
NS = 16
NW = NC * NS
CHUNK = 64
BLK = 16
NBLK0 = 14
NBLK1 = 6
C0 = NBLK0 * BLK
C1 = NBLK1 * BLK
C0TOT = NS * C0
TOT_CHUNKS = NS * (C0 + C1)
EPAD = TOT_CHUNKS * CHUNK
ACC_ROWS = 10240
TRASH = N + 16
ROWS_PER_SUB = ACC_ROWS // NS

import functools


@functools.lru_cache(maxsize=None)
def _sc_calls():
    mesh = plsc.VectorSubcoreMesh(
        core_axis_name="c", subcore_axis_name="s",
        num_cores=NC, num_subcores=NS)
    spmm_call = pl.kernel(
        _spmm_body,
        out_type=jax.ShapeDtypeStruct((NC, ACC_ROWS, H), jnp.float32),
        mesh=mesh,
        scratch_types=[
            pltpu.VMEM((BLK, CHUNK), jnp.int32),
            pltpu.VMEM((BLK, CHUNK), jnp.int32),
            pltpu.VMEM((CHUNK, H), jnp.float32),
            pltpu.VMEM((CHUNK, H), jnp.float32),
            pltpu.VMEM_SHARED((ACC_ROWS, H), jnp.float32),
            pltpu.SemaphoreType.DMA,
            pltpu.SemaphoreType.DMA,
        ],
    )
    return spmm_call


def _spmm_body(hs_hbm, srcs_hbm, dsts_hbm, out_hbm,
               sv, dv, r0, r1, acc, g0, g1):
    cid = lax.axis_index("c")
    sid = lax.axis_index("s")
    rows = (r0, r1)
    gsem = (g0, g1)

    def zrow(r, _):
        for c in range(H // 16):
            r0[r, pl.ds(c * 16, 16)] = jnp.zeros((16,), jnp.float32)
        return 0
    lax.fori_loop(0, CHUNK, zrow, 0)
    base = sid * ROWS_PER_SUB
    for k in range(ROWS_PER_SUB // CHUNK):
        pltpu.sync_copy(r0, acc.at[pl.ds(base + k * CHUNK, CHUNK)])
    plsc.subcore_barrier()

    nblk = jnp.where(cid == 0, NBLK0, NBLK1)
    start = jnp.where(cid == 0, sid * C0, C0TOT + sid * C1)

    def blk(b, _):
        cbase = pl.multiple_of(start + b * BLK, 8)
        pltpu.sync_copy(srcs_hbm.at[pl.ds(cbase, BLK)], sv)
        pltpu.sync_copy(dsts_hbm.at[pl.ds(cbase, BLK)], dv)

        def pair(qq, _):
            jb = qq * 2
            descs = [
                pltpu.async_copy(hs_hbm.at[sv.at[jb + t]], rows[t], gsem[t])
                for t in range(2)
            ]
            for t in range(2):
                descs[t].wait()
                pltpu.sync_copy(rows[t], acc.at[dv.at[jb + t]], add=True)
            return 0
        lax.fori_loop(0, BLK // 2, pair, 0)
        return 0
    lax.fori_loop(0, nblk, blk, 0)
    plsc.subcore_barrier()

    pltpu.sync_copy(acc.at[pl.ds(base, ROWS_PER_SUB)],
                    out_hbm.at[cid, pl.ds(base, ROWS_PER_SUB)])


def _lrelu(x):
    return jnp.where(x > 0, x, 0.1 * x)


def _bn_cols(x, g, be):
    m = jnp.mean(x, axis=0, keepdims=True)
    c = x - m
    v = jnp.mean(c * c, axis=0, keepdims=True)
    return c * lax.rsqrt(v + EPS) * g + be


def _prep_body(x_ref, w_ref, d0_ref, d1_ref, hs_ref, dis_ref):
    deg = 1.0 + d0_ref[...][:N, 0:1] + d1_ref[...][:N, 0:1]
    dis = lax.rsqrt(deg)
    dis_ref[...] = dis
    hs_ref[...] = jnp.dot(x_ref[...], w_ref[...],
                          preferred_element_type=jnp.float32) * dis


_prep_call = pl.pallas_call(
    _prep_body,
    out_shape=(jax.ShapeDtypeStruct((N, H), jnp.float32),
               jax.ShapeDtypeStruct((N, 1), jnp.float32)),
)


def _mid_body(a0, a1, hsp, dis, b, g, be, w, out):
    dis_v = dis[...]
    agg = (a0[...][:N] + a1[...][:N] + hsp[...]) * dis_v + b[...]
    h = _lrelu(_bn_cols(agg, g[...], be[...]))
    out[...] = jnp.dot(h, w[...], preferred_element_type=jnp.float32) * dis_v


_mid_call = pl.pallas_call(
    _mid_body,
    out_shape=jax.ShapeDtypeStruct((N, H), jnp.float32),
)


def _final_body(a0, a1, hsp, dis, b, g, be, bt_ref, fw1, fb1, g4, be4,
                fw2, fb2, out):
    agg = (a0[...][:N] + a1[...][:N] + hsp[...]) * dis[...] + b[...]
    h = _lrelu(_bn_cols(agg, g[...], be[...]))
    gi = lax.broadcasted_iota(jnp.int32, (G, N), 0)
    msk = (bt_ref[...] == gi).astype(jnp.float32)
    s = jnp.dot(msk, h, preferred_element_type=jnp.float32, precision=lax.Precision.HIGHEST)
    cnt = jnp.sum(msk, axis=1, keepdims=True)
    pooled = s / jnp.maximum(cnt, 1.0)
    z = jnp.dot(pooled, fw1[...], preferred_element_type=jnp.float32) + fb1[...]
    zl = _lrelu(_bn_cols(z, g4[...], be4[...]))
    out[...] = jnp.dot(zl, fw2[...], preferred_element_type=jnp.float32) + fb2[...]


_final_call = pl.pallas_call(
    _final_body,
    out_shape=jax.ShapeDtypeStruct((G, H), jnp.float32),
)


def kernel(x, W1, b1, g1, be1, W2, b2, g2, be2, W3, b3, g3, be3,
           fcW1, fcb1, g4, be4, fcW2, fcb2, edge_index, batch):
    src = edge_index[0]
    dst = edge_index[1]
    pad = EPAD - E
    srcs = jnp.concatenate(
        [src, jnp.zeros((pad,), jnp.int32)]).reshape(TOT_CHUNKS, CHUNK)
    dsts = jnp.concatenate(
        [dst, jnp.full((pad,), TRASH, jnp.int32)]).reshape(TOT_CHUNKS, CHUNK)

    _spmm_call = _sc_calls()
    deg = _spmm_call(jnp.ones((N, H), jnp.float32), srcs, dsts)
    hs1, dis = _prep_call(x, W1, deg[0], deg[1])

    r2 = lambda v: v.reshape(1, -1)
    p1 = _spmm_call(hs1, srcs, dsts)
    hs2 = _mid_call(p1[0], p1[1], hs1, dis, r2(b1), r2(g1), r2(be1), W2)
    p2 = _spmm_call(hs2, srcs, dsts)
    hs3 = _mid_call(p2[0], p2[1], hs2, dis, r2(b2), r2(g2), r2(be2), W3)
    p3 = _spmm_call(hs3, srcs, dsts)
    out = _final_call(p3[0], p3[1], hs3, dis, r2(b3), r2(g3), r2(be3),
                      batch.reshape(1, N), fcW1, r2(fcb1), r2(g4), r2(be4),
                      fcW2, r2(fcb2))
    return out


# speedup vs baseline: 8.0491x; 1.0370x over previous
import jax
import jax.numpy as jnp
from jax import lax
from jax.experimental import pallas as pl
from jax.experimental.pallas import tpu as pltpu
from jax.experimental.pallas import tpu_sc as plsc

N = 10000
E = 320000
H = 128
G = 64
EPS = 1e-5

NC = 2
NS = 16
NW = NC * NS
CHUNK = 64
BLK = 16
NBLK0 = 15
NBLK1 = 5
C0 = NBLK0 * BLK
C1 = NBLK1 * BLK
C0TOT = NS * C0
TOT_CHUNKS = NS * (C0 + C1)
EPAD = TOT_CHUNKS * CHUNK
ACC_ROWS = 10240
TRASH = N + 16
ROWS_PER_SUB = ACC_ROWS // NS

import functools


@functools.lru_cache(maxsize=None)
def _sc_calls():
    mesh = plsc.VectorSubcoreMesh(
        core_axis_name="c", subcore_axis_name="s",
        num_cores=NC, num_subcores=NS)
    spmm_call = pl.kernel(
        _spmm_body,
        out_type=jax.ShapeDtypeStruct((NC, ACC_ROWS, H), jnp.float32),
        mesh=mesh,
        scratch_types=[
            pltpu.VMEM((BLK, CHUNK), jnp.int32),
            pltpu.VMEM((BLK, CHUNK), jnp.int32),
            pltpu.VMEM((CHUNK, H), jnp.float32),
            pltpu.VMEM((CHUNK, H), jnp.float32),
            pltpu.VMEM_SHARED((ACC_ROWS, H), jnp.float32),
            pltpu.SemaphoreType.DMA,
            pltpu.SemaphoreType.DMA,
        ],
    )
    return spmm_call


def _spmm_body(hs_hbm, srcs_hbm, dsts_hbm, out_hbm,
               sv, dv, r0, r1, acc, g0, g1):
    cid = lax.axis_index("c")
    sid = lax.axis_index("s")
    rows = (r0, r1)
    gsem = (g0, g1)

    def zrow(r, _):
        for c in range(H // 16):
            r0[r, pl.ds(c * 16, 16)] = jnp.zeros((16,), jnp.float32)
        return 0
    lax.fori_loop(0, CHUNK, zrow, 0)
    base = sid * ROWS_PER_SUB
    for k in range(ROWS_PER_SUB // CHUNK):
        pltpu.sync_copy(r0, acc.at[pl.ds(base + k * CHUNK, CHUNK)])
    plsc.subcore_barrier()

    nblk = jnp.where(cid == 0, NBLK0, NBLK1)
    start = jnp.where(cid == 0, sid * C0, C0TOT + sid * C1)

    def blk(b, _):
        cbase = pl.multiple_of(start + b * BLK, 8)
        pltpu.sync_copy(srcs_hbm.at[pl.ds(cbase, BLK)], sv)
        pltpu.sync_copy(dsts_hbm.at[pl.ds(cbase, BLK)], dv)

        def pair(qq, _):
            jb = qq * 2
            descs = [
                pltpu.async_copy(hs_hbm.at[sv.at[jb + t]], rows[t], gsem[t])
                for t in range(2)
            ]
            for t in range(2):
                descs[t].wait()
                pltpu.sync_copy(rows[t], acc.at[dv.at[jb + t]], add=True)
            return 0
        lax.fori_loop(0, BLK // 2, pair, 0)
        return 0
    lax.fori_loop(0, nblk, blk, 0)
    plsc.subcore_barrier()

    pltpu.sync_copy(acc.at[pl.ds(base, ROWS_PER_SUB)],
                    out_hbm.at[cid, pl.ds(base, ROWS_PER_SUB)])


def _lrelu(x):
    return jnp.where(x > 0, x, 0.1 * x)


def _bn_cols(x, g, be):
    m = jnp.mean(x, axis=0, keepdims=True)
    c = x - m
    v = jnp.mean(c * c, axis=0, keepdims=True)
    return c * lax.rsqrt(v + EPS) * g + be


def _prep_body(x_ref, w_ref, d0_ref, d1_ref, hs_ref, dis_ref):
    deg = 1.0 + d0_ref[...][:N, 0:1] + d1_ref[...][:N, 0:1]
    dis = lax.rsqrt(deg)
    dis_ref[...] = dis
    hs_ref[...] = jnp.dot(x_ref[...], w_ref[...],
                          preferred_element_type=jnp.float32) * dis


_prep_call = pl.pallas_call(
    _prep_body,
    out_shape=(jax.ShapeDtypeStruct((N, H), jnp.float32),
               jax.ShapeDtypeStruct((N, 1), jnp.float32)),
)


def _mid_body(a0, a1, hsp, dis, b, g, be, w, out):
    dis_v = dis[...]
    agg = (a0[...][:N] + a1[...][:N] + hsp[...]) * dis_v + b[...]
    h = _lrelu(_bn_cols(agg, g[...], be[...]))
    out[...] = jnp.dot(h, w[...], preferred_element_type=jnp.float32) * dis_v


_mid_call = pl.pallas_call(
    _mid_body,
    out_shape=jax.ShapeDtypeStruct((N, H), jnp.float32),
)


def _final_body(a0, a1, hsp, dis, b, g, be, bt_ref, fw1, fb1, g4, be4,
                fw2, fb2, out):
    agg = (a0[...][:N] + a1[...][:N] + hsp[...]) * dis[...] + b[...]
    h = _lrelu(_bn_cols(agg, g[...], be[...]))
    gi = lax.broadcasted_iota(jnp.int32, (G, N), 0)
    msk = (bt_ref[...] == gi).astype(jnp.float32)
    s = jnp.dot(msk, h, preferred_element_type=jnp.float32, precision=lax.Precision.HIGHEST)
    cnt = jnp.sum(msk, axis=1, keepdims=True)
    pooled = s / jnp.maximum(cnt, 1.0)
    z = jnp.dot(pooled, fw1[...], preferred_element_type=jnp.float32) + fb1[...]
    zl = _lrelu(_bn_cols(z, g4[...], be4[...]))
    out[...] = jnp.dot(zl, fw2[...], preferred_element_type=jnp.float32) + fb2[...]


_final_call = pl.pallas_call(
    _final_body,
    out_shape=jax.ShapeDtypeStruct((G, H), jnp.float32),
)


def kernel(x, W1, b1, g1, be1, W2, b2, g2, be2, W3, b3, g3, be3,
           fcW1, fcb1, g4, be4, fcW2, fcb2, edge_index, batch):
    src = edge_index[0]
    dst = edge_index[1]
    pad = EPAD - E
    srcs = jnp.concatenate(
        [src, jnp.zeros((pad,), jnp.int32)]).reshape(TOT_CHUNKS, CHUNK)
    dsts = jnp.concatenate(
        [dst, jnp.full((pad,), TRASH, jnp.int32)]).reshape(TOT_CHUNKS, CHUNK)

    _spmm_call = _sc_calls()
    deg = _spmm_call(jnp.ones((N, H), jnp.float32), srcs, dsts)
    hs1, dis = _prep_call(x, W1, deg[0], deg[1])

    r2 = lambda v: v.reshape(1, -1)
    p1 = _spmm_call(hs1, srcs, dsts)
    hs2 = _mid_call(p1[0], p1[1], hs1, dis, r2(b1), r2(g1), r2(be1), W2)
    p2 = _spmm_call(hs2, srcs, dsts)
    hs3 = _mid_call(p2[0], p2[1], hs2, dis, r2(b2), r2(g2), r2(be2), W3)
    p3 = _spmm_call(hs3, srcs, dsts)
    out = _final_call(p3[0], p3[1], hs3, dis, r2(b3), r2(g3), r2(be3),
                      batch.reshape(1, N), fcW1, r2(fcb1), r2(g4), r2(be4),
                      fcW2, r2(fcb2))
    return out
